# baseline (speedup 1.0000x reference)
import functools

import jax
import jax.numpy as jnp
from jax import lax
from jax.experimental import pallas as pl
from jax.experimental.pallas import tpu as pltpu
from jax.experimental.pallas import tpu_sc as plsc

VOCAB = 100000
HIDDEN = 128
BATCH = 1024
SEQ = 200
N_IDX = BATCH * SEQ
NW = 32
PER_W = N_IDX // NW
CHUNK = 64
N_CHUNK = PER_W // CHUNK
R = 10
N_GRP = N_CHUNK // R


def _make_sc_gather():
    mesh = plsc.VectorSubcoreMesh(core_axis_name="c", subcore_axis_name="s")

    @functools.partial(
        pl.kernel,
        mesh=mesh,
        out_type=jax.ShapeDtypeStruct((N_IDX, HIDDEN), jnp.float32),
        scratch_types=[
            pltpu.VMEM((N_CHUNK, CHUNK), jnp.int32),
            pltpu.VMEM((R, CHUNK, HIDDEN), jnp.float32),
            pltpu.SemaphoreType.DMA,
            pltpu.SemaphoreType.DMA,
        ],
    )
    def gather_kernel(idx_hbm, table_hbm, out_hbm, idx_v, rows_v, sg, so):
        wid = lax.axis_index("s") * 2 + lax.axis_index("c")
        base = wid * PER_W
        pltpu.sync_copy(idx_hbm.at[wid], idx_v)

        def issue_gather(i, buf):
            pltpu.async_copy(table_hbm.at[idx_v.at[i]], rows_v.at[buf], sg)

        def wait_gather(i, buf):
            pltpu.make_async_copy(
                table_hbm.at[idx_v.at[i]], rows_v.at[buf], sg
            ).wait()

        def issue_out(i, buf):
            pltpu.async_copy(
                rows_v.at[buf], out_hbm.at[pl.ds(base + i * CHUNK, CHUNK)], so
            )

        def wait_out(i, buf):
            pltpu.make_async_copy(
                rows_v.at[buf], out_hbm.at[pl.ds(base + i * CHUNK, CHUNK)], so
            ).wait()

        for b in range(R):
            issue_gather(b, b)

        def body(j, carry):
            for b in range(R):
                i = j * R + b
                wait_gather(i, b)
                issue_out(i, b)
                @pl.when(j < N_GRP - 1)
                def _():
                    wait_out(i, b)
                    issue_gather(i + R, b)
            return carry

        lax.fori_loop(0, N_GRP, body, 0)

        for b in range(R):
            wait_out((N_GRP - 1) * R + b, b)

    return gather_kernel


_sc_gather = _make_sc_gather()


def kernel(x, word_embeddings_weight):
    idx = x.reshape(NW, N_CHUNK, CHUNK).astype(jnp.int32)
    gathered = _sc_gather(idx, word_embeddings_weight)
    return jnp.transpose(gathered.reshape(BATCH, SEQ, HIDDEN), (0, 2, 1))


# speedup vs baseline: 1.0070x; 1.0070x over previous
import functools

import jax
import jax.numpy as jnp
from jax import lax
from jax.experimental import pallas as pl
from jax.experimental.pallas import tpu as pltpu
from jax.experimental.pallas import tpu_sc as plsc

VOCAB = 100000
HIDDEN = 128
BATCH = 1024
SEQ = 200
N_IDX = BATCH * SEQ
NW = 32
PER_W = N_IDX // NW
CHUNK = 128
N_CHUNK = PER_W // CHUNK
R = 5
N_GRP = N_CHUNK // R


def _make_sc_gather():
    mesh = plsc.VectorSubcoreMesh(core_axis_name="c", subcore_axis_name="s")

    @functools.partial(
        pl.kernel,
        mesh=mesh,
        out_type=jax.ShapeDtypeStruct((N_IDX, HIDDEN), jnp.float32),
        scratch_types=[
            pltpu.VMEM((N_CHUNK, CHUNK), jnp.int32),
            pltpu.VMEM((R, CHUNK, HIDDEN), jnp.float32),
            pltpu.SemaphoreType.DMA,
            pltpu.SemaphoreType.DMA,
        ],
    )
    def gather_kernel(idx_hbm, table_hbm, out_hbm, idx_v, rows_v, sg, so):
        wid = lax.axis_index("s") * 2 + lax.axis_index("c")
        base = wid * PER_W
        pltpu.sync_copy(idx_hbm.at[wid], idx_v)

        def issue_gather(i, buf):
            pltpu.async_copy(table_hbm.at[idx_v.at[i]], rows_v.at[buf], sg)

        def wait_gather(i, buf):
            pltpu.make_async_copy(
                table_hbm.at[idx_v.at[i]], rows_v.at[buf], sg
            ).wait()

        def issue_out(i, buf):
            pltpu.async_copy(
                rows_v.at[buf], out_hbm.at[pl.ds(base + i * CHUNK, CHUNK)], so
            )

        def wait_out(i, buf):
            pltpu.make_async_copy(
                rows_v.at[buf], out_hbm.at[pl.ds(base + i * CHUNK, CHUNK)], so
            ).wait()

        for b in range(R):
            issue_gather(b, b)

        def body(j, carry):
            for b in range(R):
                i = j * R + b
                wait_gather(i, b)
                issue_out(i, b)
                @pl.when(j < N_GRP - 1)
                def _():
                    wait_out(i, b)
                    issue_gather(i + R, b)
            return carry

        lax.fori_loop(0, N_GRP, body, 0)

        for b in range(R):
            wait_out((N_GRP - 1) * R + b, b)

    return gather_kernel


_sc_gather = _make_sc_gather()


def kernel(x, word_embeddings_weight):
    idx = x.reshape(NW, N_CHUNK, CHUNK).astype(jnp.int32)
    gathered = _sc_gather(idx, word_embeddings_weight)
    return jnp.transpose(gathered.reshape(BATCH, SEQ, HIDDEN), (0, 2, 1))
